# baseline (speedup 1.0000x reference)
import functools

import jax
import jax.numpy as jnp
from jax import lax
from jax.experimental import pallas as pl
from jax.experimental.pallas import tpu as pltpu
from jax.experimental.pallas import tpu_sc as plsc

B = 1024
L = 50
EMB = 64
PHOTO_DIM = 1024
N = B * L

_NC = 2
_NS = 16
_NW = _NC * _NS
_ROWS_PER_W = N // _NW

_BLK = 800
_GRID = N // _BLK


def _sc_gather(src_flat, word_table):
    mesh = plsc.VectorSubcoreMesh(
        core_axis_name="c", subcore_axis_name="s",
        num_cores=_NC, num_subcores=_NS)

    @functools.partial(
        pl.kernel,
        out_type=jax.ShapeDtypeStruct((N, EMB), jnp.float32),
        mesh=mesh,
        scratch_types=[
            pltpu.VMEM((_ROWS_PER_W,), jnp.int32),
            pltpu.VMEM((_ROWS_PER_W, EMB), jnp.float32),
            pltpu.SemaphoreType.DMA,
        ],
        compiler_params=pltpu.CompilerParams(use_tc_tiling_on_sc=False),
    )
    def gather_k(src_hbm, table_hbm, out_hbm, idx_v, rows_v, sem):
        wid = lax.axis_index("s") * _NC + lax.axis_index("c")
        base = wid * _ROWS_PER_W
        pltpu.sync_copy(src_hbm.at[pl.ds(base, _ROWS_PER_W)], idx_v)
        pltpu.async_copy(table_hbm.at[idx_v], rows_v, sem).wait()
        pltpu.sync_copy(rows_v, out_hbm.at[pl.ds(base, _ROWS_PER_W)])

    return gather_k(src_flat, word_table)


def _tc_body(vis_ref, word_ref, seg_ref, pos_ref, W_ref, bvec_ref,
             gam_ref, bet_ref, segtab_ref, out_ref):
    x = jnp.dot(vis_ref[...], W_ref[...], preferred_element_type=jnp.float32)
    st = segtab_ref[...]
    s = seg_ref[...]
    seg_emb = jnp.where(s == 0, st[0:1, :],
                        jnp.where(s == 1, st[1:2, :], st[2:3, :]))
    total = x + word_ref[...] + pos_ref[...] + seg_emb + bvec_ref[...]
    mean = jnp.mean(total, axis=-1, keepdims=True)
    cent = total - mean
    var = jnp.mean(cent * cent, axis=-1, keepdims=True)
    out_ref[...] = cent * lax.rsqrt(var + 1e-6) * gam_ref[...] + bet_ref[...]


def _tc_fused(vis2d, word, seg2d, pos_rep, W_vis, b_vis, gamma, beta, seg_tab):
    return pl.pallas_call(
        _tc_body,
        grid=(_GRID,),
        in_specs=[
            pl.BlockSpec((_BLK, PHOTO_DIM), lambda i: (i, 0)),
            pl.BlockSpec((_BLK, EMB), lambda i: (i, 0)),
            pl.BlockSpec((_BLK, 1), lambda i: (i, 0)),
            pl.BlockSpec((_BLK, EMB), lambda i: (0, 0)),
            pl.BlockSpec((PHOTO_DIM, EMB), lambda i: (0, 0)),
            pl.BlockSpec((1, EMB), lambda i: (0, 0)),
            pl.BlockSpec((1, EMB), lambda i: (0, 0)),
            pl.BlockSpec((1, EMB), lambda i: (0, 0)),
            pl.BlockSpec((8, EMB), lambda i: (0, 0)),
        ],
        out_specs=pl.BlockSpec((_BLK, EMB), lambda i: (i, 0)),
        out_shape=jax.ShapeDtypeStruct((N, EMB), jnp.float32),
        compiler_params=pltpu.CompilerParams(
            dimension_semantics=("arbitrary",)),
    )(vis2d, word, seg2d, pos_rep, W_vis, b_vis, gamma, beta, seg_tab)


def kernel(visual, src, seg, word_table, pos_table, seg_table,
           W_vis, b_vis, ln_gamma, ln_beta):
    vis2d = visual.reshape(N, PHOTO_DIM)
    src_flat = src.reshape(N)
    seg2d = seg.reshape(N, 1)
    pos_rep = jnp.tile(pos_table[:L], (_BLK // L, 1))
    seg_tab = jnp.zeros((8, EMB), jnp.float32).at[:3].set(seg_table)
    word = _sc_gather(src_flat, word_table)
    out = _tc_fused(vis2d, word, seg2d, pos_rep, W_vis,
                    b_vis.reshape(1, EMB), ln_gamma.reshape(1, EMB),
                    ln_beta.reshape(1, EMB), seg_tab)
    return out.reshape(B, L, EMB)
